# Initial kernel scaffold; baseline (speedup 1.0000x reference)
import jax
import jax.numpy as jnp
from jax.experimental import pallas as pl


def kernel(tgt, memory, in_proj_w, in_proj_b, out_proj_w, out_proj_b, ln1_g, ln1_b, ref_w, ref_b, off_w, off_b, att_w, att_b, val_w, val_b, cout_w, cout_b, ln2_g, ln2_b, lin1_w, lin1_b, lin2_w, lin2_b, ln3_g, ln3_b):
    raise NotImplementedError("write your pallas kernel here")


# speedup vs baseline: 4.7544x; 4.7544x over previous
import functools
import math

import jax
import jax.numpy as jnp
from jax import lax
from jax.experimental import pallas as pl
from jax.experimental.pallas import tpu as pltpu
from jax.experimental.pallas import tpu_sc as plsc

B, Q, D, H, P, FF, HW = 4, 300, 256, 8, 4, 2048, 1024
DH = D // H
GW = int(HW ** 0.5)
NK = P * 4
_NC = 2
_NSUB = 16


def _ln(x, g, b):
    m = x.mean(-1, keepdims=True)
    v = ((x - m) ** 2).mean(-1, keepdims=True)
    return (x - m) / jnp.sqrt(v + 1e-5) * g + b


def _k1_body(tgt_ref, in_wT_ref, in_b_ref, out_wT_ref, out_b_ref,
             ln1g_ref, ln1b_ref, refwT_ref, refb_ref,
             offxT_ref, offbx_ref, offyT_ref, offby_ref,
             attwT_ref, attb_ref,
             x_ref, idx_ref, w_ref):
    t = tgt_ref[0]
    qkv = jnp.dot(t, in_wT_ref[...], preferred_element_type=jnp.float32)
    qkv = qkv + in_b_ref[...]
    q = qkv[:, :D]
    k = qkv[:, D:2 * D]
    v = qkv[:, 2 * D:]
    scale = 1.0 / math.sqrt(DH)
    outs = []
    for h in range(H):
        qh = q[:, h * DH:(h + 1) * DH]
        kh = k[:, h * DH:(h + 1) * DH]
        vh = v[:, h * DH:(h + 1) * DH]
        s = lax.dot_general(qh, kh, (((1,), (1,)), ((), ())),
                            preferred_element_type=jnp.float32) * scale
        p = jax.nn.softmax(s, axis=-1)
        outs.append(jnp.dot(p, vh, preferred_element_type=jnp.float32))
    sa = jnp.concatenate(outs, axis=1)
    sa = jnp.dot(sa, out_wT_ref[...], preferred_element_type=jnp.float32)
    sa = sa + out_b_ref[...]
    x = _ln(t + sa, ln1g_ref[...], ln1b_ref[...])
    x_ref[0] = x

    refxy = jax.nn.sigmoid(
        jnp.dot(x, refwT_ref[...], preferred_element_type=jnp.float32)
        + refb_ref[...])
    rx = refxy[:, 0:1]
    ry = refxy[:, 1:2]
    offx = jnp.dot(x, offxT_ref[...], preferred_element_type=jnp.float32) + offbx_ref[...]
    offy = jnp.dot(x, offyT_ref[...], preferred_element_type=jnp.float32) + offby_ref[...]
    att = jnp.dot(x, attwT_ref[...], preferred_element_type=jnp.float32) + attb_ref[...]
    a = [att[:, p * H:(p + 1) * H] for p in range(P)]
    m = jnp.maximum(jnp.maximum(a[0], a[1]), jnp.maximum(a[2], a[3]))
    e = [jnp.exp(ai - m) for ai in a]
    ssum = e[0] + e[1] + e[2] + e[3]
    wts = jnp.concatenate([ei / ssum for ei in e], axis=1)

    sx = jnp.clip(rx + offx, 0.0, 1.0) * (GW - 1)
    sy = jnp.clip(ry + offy, 0.0, 1.0) * (GW - 1)
    x0 = jnp.clip(jnp.floor(sx), 0.0, GW - 1)
    y0 = jnp.clip(jnp.floor(sy), 0.0, GW - 1)
    x1 = jnp.minimum(x0 + 1.0, GW - 1)
    y1 = jnp.minimum(y0 + 1.0, GW - 1)
    wx1 = sx - x0
    wx0 = 1.0 - wx1
    wy1 = sy - y0
    wy0 = 1.0 - wy1
    xi0 = x0.astype(jnp.int32)
    xi1 = x1.astype(jnp.int32)
    yi0 = y0.astype(jnp.int32)
    yi1 = y1.astype(jnp.int32)
    corners = ((xi0, yi0, wx0 * wy0), (xi0, yi1, wx0 * wy1),
               (xi1, yi0, wx1 * wy0), (xi1, yi1, wx1 * wy1))
    for c, (cx, cy, cw) in enumerate(corners):
        idx_ref[0, c] = cy * GW + cx
        w_ref[0, c] = wts * cw


def _k1_call(tgt, in_wT, in_b, out_wT, out_b, ln1g, ln1b, refwT, refb,
             offxT, offbx, offyT, offby, attwT, attb, interpret=False):
    full = lambda shape: pl.BlockSpec(shape, lambda b: (0,) * len(shape))
    return pl.pallas_call(
        _k1_body,
        grid=(B,),
        in_specs=[
            pl.BlockSpec((1, Q, D), lambda b: (b, 0, 0)),
            full((D, 3 * D)), full((3 * D,)),
            full((D, D)), full((D,)),
            full((D,)), full((D,)),
            full((D, 8)), full((8,)),
            full((D, H * P)), full((H * P,)),
            full((D, H * P)), full((H * P,)),
            full((D, H * P)), full((H * P,)),
        ],
        out_specs=[
            pl.BlockSpec((1, Q, D), lambda b: (b, 0, 0)),
            pl.BlockSpec((1, 4, Q, H * P), lambda b: (b, 0, 0, 0)),
            pl.BlockSpec((1, 4, Q, H * P), lambda b: (b, 0, 0, 0)),
        ],
        out_shape=[
            jax.ShapeDtypeStruct((B, Q, D), jnp.float32),
            jax.ShapeDtypeStruct((B, 4, Q, H * P), jnp.int32),
            jax.ShapeDtypeStruct((B, 4, Q, H * P), jnp.float32),
        ],
        interpret=interpret,
    )(tgt, in_wT, in_b, out_wT, out_b, ln1g, ln1b, refwT, refb,
      offxT, offbx, offyT, offby, attwT, attb)


def _k2_body(mem_ref, val_wT_ref, val_b_ref, tab_ref):
    v = jnp.dot(mem_ref[0], val_wT_ref[...],
                preferred_element_type=jnp.float32) + val_b_ref[...]
    for h in range(H):
        tab_ref[0, h] = v[:, h * DH:(h + 1) * DH]


def _k2_call(memory, val_wT, val_b, interpret=False):
    return pl.pallas_call(
        _k2_body,
        grid=(B,),
        in_specs=[
            pl.BlockSpec((1, HW, D), lambda b: (b, 0, 0)),
            pl.BlockSpec((D, D), lambda b: (0, 0)),
            pl.BlockSpec((D,), lambda b: (0,)),
        ],
        out_specs=pl.BlockSpec((1, H, HW, DH), lambda b: (b, 0, 0, 0)),
        out_shape=jax.ShapeDtypeStruct((B, H, HW, DH), jnp.float32),
        interpret=interpret,
    )(memory, val_wT, val_b)


def _sc_gather(tab, idxf, wf, interpret=False):
    mesh = plsc.VectorSubcoreMesh(core_axis_name="c", subcore_axis_name="s",
                                  num_cores=_NC, num_subcores=_NSUB)

    @functools.partial(
        pl.kernel,
        out_type=jax.ShapeDtypeStruct((B * H, Q * DH), jnp.float32),
        mesh=mesh,
        compiler_params=pltpu.CompilerParams(needs_layout_passes=False,
                                             use_tc_tiling_on_sc=False),
        scratch_types=[
            pltpu.VMEM((HW, DH), jnp.float32),
            pltpu.VMEM((Q * NK,), jnp.int32),
            pltpu.VMEM((Q * NK,), jnp.float32),
            pltpu.VMEM((Q * DH,), jnp.float32),
        ],
        interpret=interpret,
    )
    def run(tab_hbm, idx_hbm, w_hbm, out_hbm, tab_v, idx_v, w_v, out_v):
        wid = lax.axis_index("s") * _NC + lax.axis_index("c")
        pltpu.sync_copy(tab_hbm.at[wid], tab_v)
        pltpu.sync_copy(idx_hbm.at[wid], idx_v)
        pltpu.sync_copy(w_hbm.at[wid], w_v)
        lane = lax.broadcasted_iota(jnp.int32, (16,), 0)
        lane2 = lane + 16

        def body(qi, carry):
            base = qi * NK
            acc0 = jnp.zeros((16,), jnp.float32)
            acc1 = jnp.zeros((16,), jnp.float32)
            for kk in range(NK):
                sel = jnp.full((16,), base + kk, jnp.int32)
                ib = plsc.load_gather(idx_v, [sel])
                wb = plsc.load_gather(w_v, [sel])
                r0 = plsc.load_gather(tab_v, [ib, lane])
                r1 = plsc.load_gather(tab_v, [ib, lane2])
                acc0 = acc0 + wb * r0
                acc1 = acc1 + wb * r1
            ob = qi * DH
            out_v[pl.ds(ob, 16)] = acc0
            out_v[pl.ds(ob + 16, 16)] = acc1
            return carry

        lax.fori_loop(0, Q, body, 0)
        pltpu.sync_copy(out_v, out_hbm.at[wid])

    return run(tab, idxf, wf)


def _k3_body(x_ref, g_ref, cout_wT_ref, cout_b_ref, ln2g_ref, ln2b_ref,
             lin1T_ref, lin1b_ref, lin2T_ref, lin2b_ref, ln3g_ref, ln3b_ref,
             out_ref):
    ca = jnp.dot(g_ref[0], cout_wT_ref[...],
                 preferred_element_type=jnp.float32) + cout_b_ref[...]
    x2 = _ln(x_ref[0] + ca, ln2g_ref[...], ln2b_ref[...])
    h1 = jax.nn.relu(
        jnp.dot(x2, lin1T_ref[...], preferred_element_type=jnp.float32)
        + lin1b_ref[...])
    ff = jnp.dot(h1, lin2T_ref[...],
                 preferred_element_type=jnp.float32) + lin2b_ref[...]
    out_ref[0] = _ln(x2 + ff, ln3g_ref[...], ln3b_ref[...])


def _k3_call(x, g, cout_wT, cout_b, ln2g, ln2b, lin1T, lin1b, lin2T, lin2b,
             ln3g, ln3b, interpret=False):
    full = lambda shape: pl.BlockSpec(shape, lambda b: (0,) * len(shape))
    return pl.pallas_call(
        _k3_body,
        grid=(B,),
        in_specs=[
            pl.BlockSpec((1, Q, D), lambda b: (b, 0, 0)),
            pl.BlockSpec((1, Q, D), lambda b: (b, 0, 0)),
            full((D, D)), full((D,)), full((D,)), full((D,)),
            full((D, FF)), full((FF,)), full((FF, D)), full((D,)),
            full((D,)), full((D,)),
        ],
        out_specs=pl.BlockSpec((1, Q, D), lambda b: (b, 0, 0)),
        out_shape=jax.ShapeDtypeStruct((B, Q, D), jnp.float32),
        interpret=interpret,
    )(x, g, cout_wT, cout_b, ln2g, ln2b, lin1T, lin1b, lin2T, lin2b,
      ln3g, ln3b)


def kernel(tgt, memory, in_proj_w, in_proj_b, out_proj_w, out_proj_b,
           ln1_g, ln1_b, ref_w, ref_b, off_w, off_b, att_w, att_b,
           val_w, val_b, cout_w, cout_b, ln2_g, ln2_b,
           lin1_w, lin1_b, lin2_w, lin2_b, ln3_g, ln3_b):
    perm = jnp.array([(h * P + p) for p in range(P) for h in range(H)],
                     dtype=jnp.int32)
    offxT = off_w[2 * perm].T
    offyT = off_w[2 * perm + 1].T
    offbx = off_b[2 * perm]
    offby = off_b[2 * perm + 1]
    attwT = att_w[perm].T
    attb = att_b[perm]
    refwT = jnp.pad(ref_w.T, ((0, 0), (0, 6)))
    refb8 = jnp.pad(ref_b, (0, 6))

    x, idx4, w4 = _k1_call(
        tgt, in_proj_w.T, in_proj_b, out_proj_w.T, out_proj_b,
        ln1_g, ln1_b, refwT, refb8, offxT, offbx, offyT, offby, attwT, attb)

    tab = _k2_call(memory, val_w.T, val_b)

    r = idx4.reshape(B, 4, Q, P, H).transpose(0, 4, 2, 3, 1)
    idxf = r.reshape(B * H, Q * NK)
    rw = w4.reshape(B, 4, Q, P, H).transpose(0, 4, 2, 3, 1)
    wf = rw.reshape(B * H, Q * NK)

    g = _sc_gather(tab.reshape(B * H, HW, DH), idxf, wf)
    g = g.reshape(B, H, Q, DH).transpose(0, 2, 1, 3).reshape(B, Q, D)

    return _k3_call(x, g, cout_w.T, cout_b, ln2_g, ln2_b,
                    lin1_w.T, lin1_b, lin2_w.T, lin2_b, ln3_g, ln3_b)
